# baseline (speedup 1.0000x reference)
import jax
import jax.numpy as jnp
from jax.experimental import pallas as pl
from jax.experimental.pallas import tpu as pltpu

EPS = 1e-8
NTAGS = 8
SQRT_TINV = 3.1622776601683795

BM = 1024
BN = 1024
BNORM = 1024


def _normalize_kernel(x_ref, out_ref):
    x = x_ref[...]
    norm = jnp.sqrt(jnp.sum(x * x, axis=1, keepdims=True))
    scale = SQRT_TINV / jnp.maximum(norm, EPS)
    out_ref[...] = (x * scale).astype(jnp.bfloat16)


def _info_nce_kernel(nj, ni, xi_ref, xj_ref, rt_ref, ct_ref, out_ref,
                     sim_buf, r_acc, diag_acc, loss_acc):
    s = pl.program_id(0)
    cur = jax.lax.rem(s, 2)
    prv = 1 - cur

    sim_buf[cur] = jax.lax.dot_general(
        xi_ref[...], xj_ref[...], (((1,), (1,)), ((), ())),
        preferred_element_type=jnp.float32)

    sp = jnp.maximum(s - 1, 0)
    ip = sp // nj
    jp = jax.lax.rem(sp, nj)

    e_bf = jnp.exp(sim_buf[prv]).astype(jnp.bfloat16)

    ct = ct_ref[0, :]
    tag_iota = jax.lax.broadcasted_iota(jnp.int32, (BN, NTAGS), 1)
    onehot = (ct[:, None] == tag_iota).astype(jnp.bfloat16)
    r = jax.lax.dot_general(
        e_bf, onehot, (((1,), (0,)), ((), ())),
        preferred_element_type=jnp.float32)
    r_acc[...] = jnp.where(jp == 0, r, r_acc[...] + r)

    @pl.when(jp == 0)
    def _reset_diag():
        diag_acc[...] = jnp.zeros_like(diag_acc)

    @pl.when(ip == jp)
    def _diag():
        row_g = jax.lax.broadcasted_iota(jnp.int32, (BM, BN), 0)
        col_g = jax.lax.broadcasted_iota(jnp.int32, (BM, BN), 1)
        d = jnp.sum(jnp.where(row_g == col_g, e_bf.astype(jnp.float32), 0.0),
                    axis=1, keepdims=True)
        diag_acc[...] += d

    @pl.when((jp == nj - 1) & (s > 0))
    def _finalize_rows():
        rfull = r_acc[...]
        de = diag_acc[...]
        rt = rt_ref[0, :]
        sel = (rt[:, None] ==
               jax.lax.broadcasted_iota(jnp.int32, (BM, NTAGS), 1))
        den = jnp.sum(rfull, axis=1, keepdims=True) - de
        num = jnp.sum(jnp.where(sel, rfull, 0.0), axis=1, keepdims=True) - de
        valid = num > 0.0
        num_safe = jnp.where(valid, num, 1.0)
        den_safe = jnp.where(den > 0.0, den, 1.0)
        losses = -jnp.log(num_safe / den_safe)
        loss_sum = jnp.sum(jnp.where(valid, losses, 0.0))
        cnt = jnp.sum(valid.astype(jnp.float32))

        @pl.when(ip == 0)
        def _():
            loss_acc[0, 0] = loss_sum
            loss_acc[0, 1] = cnt

        @pl.when(ip != 0)
        def _():
            loss_acc[0, 0] += loss_sum
            loss_acc[0, 1] += cnt

        @pl.when(ip == ni - 1)
        def _():
            out_ref[0, 0] = loss_acc[0, 0] / jnp.maximum(loss_acc[0, 1], 1.0)


def kernel(representations, ne_tags):
    n, d = representations.shape
    tags = ne_tags.astype(jnp.int32).reshape(1, n)

    xn = pl.pallas_call(
        _normalize_kernel,
        grid=(n // BNORM,),
        in_specs=[pl.BlockSpec((BNORM, d), lambda i: (i, 0))],
        out_specs=pl.BlockSpec((BNORM, d), lambda i: (i, 0)),
        out_shape=jax.ShapeDtypeStruct((n, d), jnp.bfloat16),
    )(representations)

    ni = n // BM
    nj = n // BN
    steps = ni * nj + 1

    def body(*refs):
        _info_nce_kernel(nj, ni, *refs)

    out = pl.pallas_call(
        body,
        grid=(steps,),
        in_specs=[
            pl.BlockSpec((BM, d), lambda s: (jnp.minimum(s // nj, ni - 1), 0)),
            pl.BlockSpec((BN, d), lambda s: (jax.lax.rem(s, nj), 0)),
            pl.BlockSpec((1, BM), lambda s: (0, jnp.maximum(s - 1, 0) // nj)),
            pl.BlockSpec((1, BN),
                         lambda s: (0, jax.lax.rem(jnp.maximum(s - 1, 0), nj))),
        ],
        out_specs=pl.BlockSpec(
            (1, 2), lambda s: (0, 0), memory_space=pltpu.SMEM),
        out_shape=jax.ShapeDtypeStruct((1, 2), jnp.float32),
        scratch_shapes=[
            pltpu.VMEM((2, BM, BN), jnp.float32),
            pltpu.VMEM((BM, NTAGS), jnp.float32),
            pltpu.VMEM((BM, 1), jnp.float32),
            pltpu.SMEM((1, 2), jnp.float32),
        ],
        compiler_params=pltpu.CompilerParams(
            dimension_semantics=("arbitrary",)),
    )(xn, xn, tags, tags)
    return out[0, 0]


# speedup vs baseline: 1.1353x; 1.1353x over previous
import jax
import jax.numpy as jnp
from jax.experimental import pallas as pl
from jax.experimental.pallas import tpu as pltpu

EPS = 1e-8
NTAGS = 8
SQRT_TINV = 3.1622776601683795

BM = 1024
BN = 1024
BNORM = 1024


def _normalize_kernel(x_ref, out_ref):
    x = x_ref[...]
    norm = jnp.sqrt(jnp.sum(x * x, axis=1, keepdims=True))
    scale = SQRT_TINV / jnp.maximum(norm, EPS)
    out_ref[...] = (x * scale).astype(jnp.bfloat16)


def _sim(xi_ref, xj_ref):
    return jax.lax.dot_general(
        xi_ref[...], xj_ref[...], (((1,), (1,)), ((), ())),
        preferred_element_type=jnp.float32)


def _info_nce_body(nj, ni, xi_ref, xja_ref, xjb_ref, ct1_ref, ct2_ref,
                   ct3_ref, rt1_ref, rt3_ref, out_ref,
                   buf_a, buf_b, r_acc, diag_acc, loss_acc):
    t = pl.program_id(0)
    nt = pl.num_programs(0)
    tag_iota_c = jax.lax.broadcasted_iota(jnp.int32, (BN, NTAGS), 1)
    tag_iota_r = jax.lax.broadcasted_iota(jnp.int32, (BM, NTAGS), 1)

    def reduce_tile(buf, ct_ref):
        e_bf = jnp.exp(buf[...]).astype(jnp.bfloat16)
        onehot = (ct_ref[0, :][:, None] == tag_iota_c).astype(jnp.bfloat16)
        r = jax.lax.dot_general(
            e_bf, onehot, (((1,), (0,)), ((), ())),
            preferred_element_type=jnp.float32)
        return e_bf, r

    def accumulate(tile, e_bf, r):
        ip = tile // nj
        jp = jax.lax.rem(tile, nj)
        r_acc[...] = jnp.where(jp == 0, r, r_acc[...] + r)

        @pl.when(jp == 0)
        def _reset_diag():
            diag_acc[...] = jnp.zeros_like(diag_acc)

        @pl.when(ip == jp)
        def _diag():
            row_g = jax.lax.broadcasted_iota(jnp.int32, (BM, BN), 0)
            col_g = jax.lax.broadcasted_iota(jnp.int32, (BM, BN), 1)
            d = jnp.sum(
                jnp.where(row_g == col_g, e_bf.astype(jnp.float32), 0.0),
                axis=1, keepdims=True)
            diag_acc[...] += d

    def finalize(tile, rt_ref):
        ip = tile // nj
        jp = jax.lax.rem(tile, nj)

        @pl.when(jp == nj - 1)
        def _finalize_rows():
            rfull = r_acc[...]
            de = diag_acc[...]
            sel = rt_ref[0, :][:, None] == tag_iota_r
            den = jnp.sum(rfull, axis=1, keepdims=True) - de
            num = jnp.sum(jnp.where(sel, rfull, 0.0),
                          axis=1, keepdims=True) - de
            valid = num > 0.0
            num_safe = jnp.where(valid, num, 1.0)
            den_safe = jnp.where(den > 0.0, den, 1.0)
            losses = -jnp.log(num_safe / den_safe)
            loss_sum = jnp.sum(jnp.where(valid, losses, 0.0))
            cnt = jnp.sum(valid.astype(jnp.float32))

            @pl.when(ip == 0)
            def _():
                loss_acc[0, 0] = loss_sum
                loss_acc[0, 1] = cnt

            @pl.when(ip != 0)
            def _():
                loss_acc[0, 0] += loss_sum
                loss_acc[0, 1] += cnt

            @pl.when(ip == ni - 1)
            def _():
                out_ref[0, 0] = loss_acc[0, 0] / jnp.maximum(
                    loss_acc[0, 1], 1.0)

    buf_a[...] = _sim(xi_ref, xja_ref)

    p1 = jnp.maximum(2 * t - 1, 0)
    e1, r1 = reduce_tile(buf_b, ct1_ref)

    buf_b[...] = _sim(xi_ref, xjb_ref)

    accumulate(p1, e1, r1)
    finalize(p1, rt1_ref)

    p2 = 2 * t
    e2, r2 = reduce_tile(buf_a, ct2_ref)
    accumulate(p2, e2, r2)

    @pl.when(t == nt - 1)
    def _tail():
        p3 = 2 * t + 1
        e3, r3 = reduce_tile(buf_b, ct3_ref)
        accumulate(p3, e3, r3)
        finalize(p3, rt3_ref)


def kernel(representations, ne_tags):
    n, d = representations.shape
    tags = ne_tags.astype(jnp.int32).reshape(1, n)

    xn = pl.pallas_call(
        _normalize_kernel,
        grid=(n // BNORM,),
        in_specs=[pl.BlockSpec((BNORM, d), lambda i: (i, 0))],
        out_specs=pl.BlockSpec((BNORM, d), lambda i: (i, 0)),
        out_shape=jax.ShapeDtypeStruct((n, d), jnp.bfloat16),
    )(representations)

    ni = n // BM
    nj = n // BN
    assert (ni * nj) % 2 == 0 and nj % 2 == 0

    def body(*refs):
        _info_nce_body(nj, ni, *refs)

    rem = jax.lax.rem
    out = pl.pallas_call(
        body,
        grid=(ni * nj // 2,),
        in_specs=[
            pl.BlockSpec((BM, d), lambda s: ((2 * s) // nj, 0)),
            pl.BlockSpec((BN, d), lambda s: (rem(2 * s, nj), 0)),
            pl.BlockSpec((BN, d), lambda s: (rem(2 * s + 1, nj), 0)),
            pl.BlockSpec((1, BN),
                         lambda s: (0, rem(jnp.maximum(2 * s - 1, 0), nj))),
            pl.BlockSpec((1, BN), lambda s: (0, rem(2 * s, nj))),
            pl.BlockSpec((1, BN), lambda s: (0, rem(2 * s + 1, nj))),
            pl.BlockSpec((1, BM),
                         lambda s: (0, jnp.maximum(2 * s - 1, 0) // nj)),
            pl.BlockSpec((1, BM), lambda s: (0, (2 * s + 1) // nj)),
        ],
        out_specs=pl.BlockSpec(
            (1, 2), lambda s: (0, 0), memory_space=pltpu.SMEM),
        out_shape=jax.ShapeDtypeStruct((1, 2), jnp.float32),
        scratch_shapes=[
            pltpu.VMEM((BM, BN), jnp.float32),
            pltpu.VMEM((BM, BN), jnp.float32),
            pltpu.VMEM((BM, NTAGS), jnp.float32),
            pltpu.VMEM((BM, 1), jnp.float32),
            pltpu.SMEM((1, 2), jnp.float32),
        ],
        compiler_params=pltpu.CompilerParams(
            dimension_semantics=("arbitrary",)),
    )(xn, xn, xn, tags, tags, tags, tags, tags)
    return out[0, 0]


# speedup vs baseline: 1.2133x; 1.0687x over previous
import jax
import jax.numpy as jnp
from jax.experimental import pallas as pl
from jax.experimental.pallas import tpu as pltpu

EPS = 1e-8
NTAGS = 8
SQRT_TINV = 3.1622776601683795

BM = 1024
BN = 1024
BNORM = 1024


def _normalize_kernel(x_ref, out_ref):
    x = x_ref[...]
    norm = jnp.sqrt(jnp.sum(x * x, axis=1, keepdims=True))
    scale = SQRT_TINV / jnp.maximum(norm, EPS)
    out_ref[...] = (x * scale).astype(jnp.bfloat16)


def _info_nce_body(nj, ni, xi_ref, xj_ref, ct_ref, rt_ref, out_ref,
                   buf_a, buf_b, r_acc, diag_acc, loss_acc):
    s = pl.program_id(0)
    tag_iota_c = jax.lax.broadcasted_iota(jnp.int32, (BN, NTAGS), 1)
    tag_iota_r = jax.lax.broadcasted_iota(jnp.int32, (BM, NTAGS), 1)

    p = jnp.maximum(s - 1, 0)
    ip = p // nj
    jp = jax.lax.rem(p, nj)

    def step(dst_ref, src_ref):
        dst_ref[...] = jax.lax.dot_general(
            xi_ref[...], xj_ref[...], (((1,), (1,)), ((), ())),
            preferred_element_type=jnp.float32)

        e_bf = jnp.exp(src_ref[...]).astype(jnp.bfloat16)
        onehot = (ct_ref[0, :][:, None] == tag_iota_c).astype(jnp.bfloat16)
        r = jax.lax.dot_general(
            e_bf, onehot, (((1,), (0,)), ((), ())),
            preferred_element_type=jnp.float32)
        r_acc[...] = jnp.where(jp == 0, r, r_acc[...] + r)

        @pl.when(jp == 0)
        def _reset_diag():
            diag_acc[...] = jnp.zeros_like(diag_acc)

        @pl.when(ip == jp)
        def _diag():
            row_g = jax.lax.broadcasted_iota(jnp.int32, (BM, BN), 0)
            col_g = jax.lax.broadcasted_iota(jnp.int32, (BM, BN), 1)
            d = jnp.sum(
                jnp.where(row_g == col_g, e_bf.astype(jnp.float32), 0.0),
                axis=1, keepdims=True)
            diag_acc[...] += d

        @pl.when((jp == nj - 1) & (s > 0))
        def _finalize_rows():
            rfull = r_acc[...]
            de = diag_acc[...]
            sel = rt_ref[0, :][:, None] == tag_iota_r
            den = jnp.sum(rfull, axis=1, keepdims=True) - de
            num = jnp.sum(jnp.where(sel, rfull, 0.0),
                          axis=1, keepdims=True) - de
            valid = num > 0.0
            num_safe = jnp.where(valid, num, 1.0)
            den_safe = jnp.where(den > 0.0, den, 1.0)
            losses = -jnp.log(num_safe / den_safe)
            loss_sum = jnp.sum(jnp.where(valid, losses, 0.0))
            cnt = jnp.sum(valid.astype(jnp.float32))

            @pl.when(ip == 0)
            def _():
                loss_acc[0, 0] = loss_sum
                loss_acc[0, 1] = cnt

            @pl.when(ip != 0)
            def _():
                loss_acc[0, 0] += loss_sum
                loss_acc[0, 1] += cnt

            @pl.when(ip == ni - 1)
            def _():
                out_ref[0, 0] = loss_acc[0, 0] / jnp.maximum(
                    loss_acc[0, 1], 1.0)

    @pl.when(jax.lax.rem(s, 2) == 0)
    def _even():
        step(buf_a, buf_b)

    @pl.when(jax.lax.rem(s, 2) == 1)
    def _odd():
        step(buf_b, buf_a)


def kernel(representations, ne_tags):
    n, d = representations.shape
    tags = ne_tags.astype(jnp.int32).reshape(1, n)

    xn = pl.pallas_call(
        _normalize_kernel,
        grid=(n // BNORM,),
        in_specs=[pl.BlockSpec((BNORM, d), lambda i: (i, 0))],
        out_specs=pl.BlockSpec((BNORM, d), lambda i: (i, 0)),
        out_shape=jax.ShapeDtypeStruct((n, d), jnp.bfloat16),
    )(representations)

    ni = n // BM
    nj = n // BN

    def body(*refs):
        _info_nce_body(nj, ni, *refs)

    rem = jax.lax.rem
    out = pl.pallas_call(
        body,
        grid=(ni * nj + 1,),
        in_specs=[
            pl.BlockSpec((BM, d), lambda s: (jnp.minimum(s // nj, ni - 1), 0)),
            pl.BlockSpec((BN, d), lambda s: (rem(s, nj), 0)),
            pl.BlockSpec((1, BN),
                         lambda s: (0, rem(jnp.maximum(s - 1, 0), nj))),
            pl.BlockSpec((1, BM),
                         lambda s: (0, jnp.maximum(s - 1, 0) // nj)),
        ],
        out_specs=pl.BlockSpec(
            (1, 2), lambda s: (0, 0), memory_space=pltpu.SMEM),
        out_shape=jax.ShapeDtypeStruct((1, 2), jnp.float32),
        scratch_shapes=[
            pltpu.VMEM((BM, BN), jnp.float32),
            pltpu.VMEM((BM, BN), jnp.float32),
            pltpu.VMEM((BM, NTAGS), jnp.float32),
            pltpu.VMEM((BM, 1), jnp.float32),
            pltpu.SMEM((1, 2), jnp.float32),
        ],
        compiler_params=pltpu.CompilerParams(
            dimension_semantics=("arbitrary",)),
    )(xn, xn, tags, tags)
    return out[0, 0]
